# baseline (speedup 1.0000x reference)
import functools

import jax
import jax.numpy as jnp
from jax import lax
from jax.experimental import pallas as pl
from jax.experimental.pallas import tpu as pltpu
from jax.experimental.pallas import tpu_sc as plsc

B = 16384
DA = 10
DB = 50
DP = 64
G = 4
DQ = G * DP
NQ = 4 ** G
NC = 2
NS = 16
NW = NC * NS
QPW = (B // G) // NW

_MESH = plsc.VectorSubcoreMesh(core_axis_name="c", subcore_axis_name="s")


@functools.partial(
    pl.kernel,
    mesh=_MESH,
    out_type=jax.ShapeDtypeStruct((B // G, DQ), jnp.float32),
    scratch_types=[
        pltpu.VMEM((QPW,), jnp.int32),
        pltpu.VMEM((QPW, DQ), jnp.float32),
        pltpu.SemaphoreType.DMA,
    ],
)
def _sc_gather(qc_hbm, tab_hbm, out_hbm, qc_v, rows_v, sem):
    wid = lax.axis_index("s") * NC + lax.axis_index("c")
    pltpu.sync_copy(qc_hbm.at[wid], qc_v)
    pltpu.async_copy(tab_hbm.at[qc_v], rows_v, sem).wait()
    pltpu.sync_copy(rows_v, out_hbm.at[pl.ds(wid * QPW, QPW)])


def kernel(index, a, b):
    idx = index.astype(jnp.int32).reshape(B // G, G)
    qc = ((idx[:, 0] * 4 + idx[:, 1]) * 4 + idx[:, 2]) * 4 + idx[:, 3]
    qc = qc.reshape(NW, QPW)
    tab = jnp.concatenate(
        [a.reshape(4, DA), b.reshape(4, DB),
         jnp.zeros((4, DP - DA - DB), jnp.float32)], axis=1)
    q = jnp.arange(NQ, dtype=jnp.int32)
    digits = jnp.stack(
        [(q >> (2 * (G - 1 - c))) & 3 for c in range(G)], axis=1)
    tab_quad = tab[digits].reshape(NQ, DQ)
    out_q = _sc_gather(qc, tab_quad)
    out_a = jnp.concatenate(
        [out_q[:, DP * c:DP * c + DA] for c in range(G)], axis=1)
    out_b = jnp.concatenate(
        [out_q[:, DP * c + DA:DP * c + DA + DB] for c in range(G)], axis=1)
    return (out_a.reshape(B, 2, 5), out_b.reshape(B, 2, 5, 5))


# speedup vs baseline: 6.1743x; 6.1743x over previous
import functools

import jax
import jax.numpy as jnp
from jax import lax
from jax.experimental import pallas as pl
from jax.experimental.pallas import tpu as pltpu
from jax.experimental.pallas import tpu_sc as plsc

B = 16384
DA = 10
DB = 50
DC = DA + DB
NC = 2
NS = 16
NW = NC * NS
BPW = B // NW
NH = BPW // 128

_MESH = plsc.VectorSubcoreMesh(core_axis_name="c", subcore_axis_name="s")


@functools.partial(
    pl.kernel,
    mesh=_MESH,
    out_type=(
        jax.ShapeDtypeStruct((5 * B * 2,), jnp.float32),
        jax.ShapeDtypeStruct((25 * B * 2,), jnp.float32),
    ),
    scratch_types=[
        pltpu.VMEM((BPW,), jnp.int32),
        pltpu.VMEM((DC, 4, 16), jnp.float32),
        pltpu.VMEM((5 * NH * 256,), jnp.float32),
        pltpu.VMEM((25 * NH * 256,), jnp.float32),
        pltpu.SemaphoreType.DMA,
    ],
)
def _sc_select(idx_hbm, tab_hbm, out_a, out_b, idx_v, tab_v, buf_a, buf_b,
               sem):
    wid = lax.axis_index("s") * NC + lax.axis_index("c")
    pltpu.sync_copy(idx_hbm.at[wid], idx_v)
    pltpu.sync_copy(tab_hbm, tab_v)
    for ihh in range(NH):
        def body(g, carry):
            o = g * 16
            v = idx_v[pl.ds(ihh * 128 + o, 16)]
            b0 = (v & 1) == 1
            b1 = v >= 2
            for c in range(DC):
                s0 = tab_v[c, 0]
                s1 = tab_v[c, 1]
                s2 = tab_v[c, 2]
                s3 = tab_v[c, 3]
                val = jnp.where(b1, jnp.where(b0, s3, s2),
                                jnp.where(b0, s1, s0))
                if c < DA:
                    d1, d2 = divmod(c, 5)
                    base = (d2 * NH + ihh) * 256 + d1 * 128
                    buf_a[pl.ds(base + o, 16)] = val
                else:
                    d1, r = divmod(c - DA, 25)
                    d2, d3 = divmod(r, 5)
                    base = ((d2 * 5 + d3) * NH + ihh) * 256 + d1 * 128
                    buf_b[pl.ds(base + o, 16)] = val
            return carry
        lax.fori_loop(0, 128 // 16, body, 0)
    run = NH * 256
    copies = []
    for d2 in range(5):
        copies.append(pltpu.async_copy(
            buf_a.at[pl.ds(d2 * run, run)],
            out_a.at[pl.ds(d2 * (B * 2) + wid * run, run)], sem))
    for t in range(25):
        copies.append(pltpu.async_copy(
            buf_b.at[pl.ds(t * run, run)],
            out_b.at[pl.ds(t * (B * 2) + wid * run, run)], sem))
    for cp in copies:
        cp.wait()


def kernel(index, a, b):
    idx = index.astype(jnp.int32).reshape(NW, BPW)
    t60 = jnp.concatenate([a.reshape(4, DA), b.reshape(4, DB)], axis=1)
    tabrep = jnp.broadcast_to(t60.T[:, :, None], (DC, 4, 16))
    ka, kb = _sc_select(idx, tabrep)
    out_a = ka.reshape(5, 128, 2, 128).transpose(1, 3, 2, 0).reshape(B, 2, 5)
    out_b = kb.reshape(5, 5, 128, 2, 128).transpose(
        2, 4, 3, 0, 1).reshape(B, 2, 5, 5)
    return (out_a, out_b)


# speedup vs baseline: 8.2728x; 1.3399x over previous
import functools

import jax
import jax.numpy as jnp
from jax import lax
from jax.experimental import pallas as pl
from jax.experimental.pallas import tpu as pltpu
from jax.experimental.pallas import tpu_sc as plsc

B = 16384
DA = 10
DB = 50
DC = DA + DB
NC = 2
NS = 16
NW = NC * NS
BPW = B // NW
NH = BPW // 128

_MESH = plsc.VectorSubcoreMesh(core_axis_name="c", subcore_axis_name="s")


@functools.partial(
    pl.kernel,
    mesh=_MESH,
    out_type=(
        jax.ShapeDtypeStruct((5 * B * 2,), jnp.float32),
        jax.ShapeDtypeStruct((25 * B * 2,), jnp.float32),
    ),
    scratch_types=[
        pltpu.VMEM((BPW,), jnp.int32),
        pltpu.VMEM((DC, 4, 16), jnp.float32),
        pltpu.VMEM((5 * NH * 256,), jnp.float32),
        pltpu.VMEM((25 * NH * 256,), jnp.float32),
        pltpu.SemaphoreType.DMA,
    ],
)
def _sc_select(idx_hbm, tab_hbm, out_a, out_b, idx_v, tab_v, buf_a, buf_b,
               sem):
    wid = lax.axis_index("s") * NC + lax.axis_index("c")
    pltpu.sync_copy(idx_hbm.at[wid], idx_v)
    pltpu.sync_copy(tab_hbm, tab_v)
    for cc in range(DC // 10):
        cols = list(range(cc * 10, cc * 10 + 10))
        svec = [[tab_v[c, k] for k in range(4)] for c in cols]
        for ihh in range(NH):
            def body(g, carry):
                o = g * 16
                v = idx_v[pl.ds(ihh * 128 + o, 16)]
                b0 = (v & 1) == 1
                b1 = v >= 2
                for ci, c in enumerate(cols):
                    s0, s1, s2, s3 = svec[ci]
                    val = jnp.where(b1, jnp.where(b0, s3, s2),
                                    jnp.where(b0, s1, s0))
                    if c < DA:
                        d1, d2 = divmod(c, 5)
                        base = (d2 * NH + ihh) * 256 + d1 * 128
                        buf_a[pl.ds(base + o, 16)] = val
                    else:
                        d1, r = divmod(c - DA, 25)
                        d2, d3 = divmod(r, 5)
                        base = ((d2 * 5 + d3) * NH + ihh) * 256 + d1 * 128
                        buf_b[pl.ds(base + o, 16)] = val
                return carry
            lax.fori_loop(0, 128 // 16, body, 0)
    run = NH * 256
    copies = []
    for d2 in range(5):
        copies.append(pltpu.async_copy(
            buf_a.at[pl.ds(d2 * run, run)],
            out_a.at[pl.ds(d2 * (B * 2) + wid * run, run)], sem))
    for t in range(25):
        copies.append(pltpu.async_copy(
            buf_b.at[pl.ds(t * run, run)],
            out_b.at[pl.ds(t * (B * 2) + wid * run, run)], sem))
    for cp in copies:
        cp.wait()


def kernel(index, a, b):
    idx = index.astype(jnp.int32).reshape(NW, BPW)
    t60 = jnp.concatenate([a.reshape(4, DA), b.reshape(4, DB)], axis=1)
    tabrep = jnp.broadcast_to(t60.T[:, :, None], (DC, 4, 16))
    ka, kb = _sc_select(idx, tabrep)
    out_a = ka.reshape(5, 128, 2, 128).transpose(1, 3, 2, 0).reshape(B, 2, 5)
    out_b = kb.reshape(5, 5, 128, 2, 128).transpose(
        2, 4, 3, 0, 1).reshape(B, 2, 5, 5)
    return (out_a, out_b)
